# baseline (speedup 1.0000x reference)
import functools

import jax
import jax.numpy as jnp
from jax import lax
from jax.experimental import pallas as pl
from jax.experimental.pallas import tpu as pltpu
from jax.experimental.pallas import tpu_sc as plsc

B, C, N = 4, 256, 2048
NUM_BINS = 8
M = 256
K_NN = 32
Q_OUT = 256
CS = N // NUM_BINS
NT = N + NUM_BINS


def _sortable_key(f):
    u = lax.bitcast_convert_type(f, jnp.int32)
    int_min = jnp.asarray(-2147483648, jnp.int32)
    return jnp.where(u >= 0, u, (int_min - u) - 1)


def _fused_body(attn_ref, xyzt_ref, xyz_ref, sqc_ref, sqr_ref, acc_ref):
    i = pl.program_id(1)
    inner = lax.dot_general(xyzt_ref[0], xyz_ref[0], (((1,), (0,)), ((), ())),
                            preferred_element_type=jnp.float32)
    d2b = (sqc_ref[0] + sqr_ref[0]) - 2.0 * inner
    key = _sortable_key(d2b)
    R = key.shape[0]
    cols = lax.broadcasted_iota(jnp.int32, (R, N), 1)
    g = lax.broadcasted_iota(jnp.int32, (R, 1), 0) + i * R
    int_max = jnp.asarray(2147483647, jnp.int32)
    keyx = jnp.where(cols == g, int_max, key)
    lo = jnp.min(keyx, axis=1, keepdims=True)
    m128 = key[:, 0:128]
    for t in range(1, 16):
        m128 = jnp.minimum(m128, key[:, t * 128:(t + 1) * 128])
    hi = jnp.max(m128, axis=1, keepdims=True)
    hi = jnp.maximum(hi, lo)

    kh = (key >> 16).astype(jnp.int16)
    kl = ((key & 0xFFFF) - 32768).astype(jnp.int16)

    def count16(b):
        c = b.astype(jnp.int16)
        w = c.shape[1] // 2
        while w >= 8:
            c = c[:, :w] + c[:, w:2 * w]
            w //= 2
        return jnp.sum(c.astype(jnp.int32), axis=1, keepdims=True)

    def bs_cond(carry):
        lo, hi = carry
        return jnp.any(lo < hi)

    def bs_body_h(carry):
        lo, hi = carry
        mid = (lo & hi) + ((lo ^ hi) >> 1)
        ge = count16(kh <= mid.astype(jnp.int16)) >= K_NN
        return jnp.where(ge, lo, mid + 1), jnp.where(ge, mid, hi)

    _, h_star = lax.while_loop(bs_cond, bs_body_h, (lo >> 16, hi >> 16))
    h16 = h_star.astype(jnp.int16)
    eqh = kh == h16
    need = K_NN - count16(kh < h16)

    def bs_body_l(carry):
        lo, hi = carry
        mid = (lo & hi) + ((lo ^ hi) >> 1)
        ge = count16(eqh & (kl <= mid.astype(jnp.int16))) >= need
        return jnp.where(ge, lo, mid + 1), jnp.where(ge, mid, hi)

    lo_l = jnp.full((R, 1), jnp.int32(-32768))
    hi_l = jnp.full((R, 1), jnp.int32(32767))
    _, l_star = lax.while_loop(bs_cond, bs_body_l, (lo_l, hi_l))
    t = (h_star << 16) | (l_star + 32768)
    cnt_t = jnp.sum((key <= t).astype(jnp.int32), axis=1, keepdims=True)

    def with_ties(_):
        c_less = jnp.sum((key < t).astype(jnp.int32), axis=1, keepdims=True)
        t_allowed = K_NN - c_less
        tie = key == t
        tlo = jnp.zeros((R, 1), jnp.int32)
        thi = jnp.full((R, 1), jnp.int32(N - 1))

        def ts_body(_, carry):
            tlo, thi = carry
            mid = tlo + ((thi - tlo) >> 1)
            cnt = jnp.sum((tie & (cols <= mid)).astype(jnp.int32), axis=1,
                          keepdims=True)
            ge = cnt >= t_allowed
            return jnp.where(ge, tlo, mid + 1), jnp.where(ge, mid, thi)

        _, thi = lax.fori_loop(0, 11, ts_body, (tlo, thi))
        return thi

    def no_ties(_):
        return jnp.full((R, 1), jnp.int32(N - 1))

    T = lax.cond(jnp.any(cnt_t > K_NN), with_ties, no_ties, 0)
    maskf = ((key < t) | ((key == t) & (cols <= T))).astype(jnp.float32)
    prod = attn_ref[0, :, :N] * maskf

    @pl.when(i == 0)
    def _():
        acc_ref[...] = jnp.zeros_like(acc_ref)

    acc = acc_ref[0, 0]
    for t2 in range(32):
        acc = acc + prod[t2 * 8:(t2 + 1) * 8, :]
    acc_ref[0, 0] = acc


def _fused_aps_pallas(attn, x_xyz):
    sq = jnp.sum(x_xyz * x_xyz, axis=1)
    xyzT = jnp.swapaxes(x_xyz, 1, 2)
    acc8 = pl.pallas_call(
        _fused_body,
        grid=(B, 8),
        in_specs=[pl.BlockSpec((1, 256, NT), lambda b, i: (b, i, 0)),
                  pl.BlockSpec((1, 256, 3), lambda b, i: (b, i, 0)),
                  pl.BlockSpec((1, 3, N), lambda b, i: (b, 0, 0)),
                  pl.BlockSpec((1, 256, 1), lambda b, i: (b, i, 0)),
                  pl.BlockSpec((1, 1, N), lambda b, i: (b, 0, 0))],
        out_specs=pl.BlockSpec((1, 1, 8, N), lambda b, i: (b, 0, 0, 0)),
        out_shape=jax.ShapeDtypeStruct((B, 1, 8, N), jnp.float32),
    )(attn, xyzT, x_xyz, sq[:, :, None], sq[:, None, :])[:, 0]
    s = acc8
    t_ = s[:, 0:4] + s[:, 4:8]
    t_ = t_[:, 0:2] + t_[:, 2:4]
    return t_[:, 0] + t_[:, 1]


def _select_body(aps_row_ref, aps_col_ref, kscum_ref, idx_ref):
    j = pl.program_id(1)
    a_row = aps_row_ref[0]
    a_col = aps_col_ref[0]
    ksrow = kscum_ref[0]
    RB = a_col.shape[0]
    cols = lax.broadcasted_iota(jnp.int32, (RB, N), 1)
    rows = lax.broadcasted_iota(jnp.int32, (RB, 1), 0)
    m_col = rows + j * RB
    gt = (a_row > a_col).astype(jnp.int32)
    tie = ((a_row == a_col) & (cols < m_col)).astype(jnp.int32)
    p = jnp.sum(gt + tie, axis=1, keepdims=True)
    j_bin = p >> 8
    pos = p & jnp.int32(CS - 1)
    t16 = lax.broadcasted_iota(jnp.int32, (RB, 16), 1)
    kscum_j = jnp.sum(jnp.where(t16 == j_bin, ksrow, 0), axis=1, keepdims=True)
    kscum_j1 = jnp.sum(jnp.where(t16 == j_bin + 1, ksrow, 0), axis=1,
                       keepdims=True)
    flag = pos < (kscum_j1 - kscum_j)
    slot = kscum_j + pos
    total = jnp.sum(jnp.where(lax.broadcasted_iota(jnp.int32, (1, 16), 1) == 8,
                              kscum_ref[0], 0), axis=1, keepdims=True)
    s_iota = lax.broadcasted_iota(jnp.int32, (RB, M), 1)
    sel = flag & (s_iota == slot)
    pad = s_iota == (p + total)
    contrib = jnp.sum(m_col * (sel.astype(jnp.int32) + pad.astype(jnp.int32)),
                      axis=0, keepdims=True)

    @pl.when(j == 0)
    def _():
        idx_ref[0] = jnp.zeros_like(idx_ref[0])

    idx_ref[0] += contrib


def _select_pallas(aps, kscum16):
    RB = 256
    aps_row = aps[:, None, :]
    aps_col = aps[:, :, None]
    idx = pl.pallas_call(
        _select_body,
        grid=(B, N // RB),
        in_specs=[
            pl.BlockSpec((1, 1, N), lambda b, i: (b, 0, 0)),
            pl.BlockSpec((1, RB, 1), lambda b, i: (b, i, 0)),
            pl.BlockSpec((1, 1, 16), lambda b, i: (b, 0, 0)),
        ],
        out_specs=pl.BlockSpec((1, 1, M), lambda b, i: (b, 0, 0)),
        out_shape=jax.ShapeDtypeStruct((B, 1, M), jnp.int32),
    )(aps_row, aps_col, kscum16)
    return idx[:, 0, :]


def _sc_gather_rows(table, idx):
    n_rows, depth = idx.shape[0], table.shape[1]
    info = plsc.get_sparse_core_info()
    nw = info.num_cores * info.num_subcores
    per_w = n_rows // nw
    mesh = plsc.VectorSubcoreMesh(core_axis_name="c", subcore_axis_name="s")

    @functools.partial(
        pl.kernel, mesh=mesh,
        out_type=jax.ShapeDtypeStruct((n_rows, depth), jnp.float32),
        scratch_types=[
            pltpu.VMEM((per_w,), jnp.int32),
            pltpu.VMEM((per_w, depth), jnp.float32),
            pltpu.SemaphoreType.DMA,
        ],
    )
    def k(table_hbm, idx_hbm, out_hbm, idx_v, rows_v, sem):
        wid = lax.axis_index("s") * info.num_cores + lax.axis_index("c")
        base = wid * per_w
        pltpu.sync_copy(idx_hbm.at[pl.ds(base, per_w)], idx_v)
        pltpu.async_copy(table_hbm.at[idx_v], rows_v, sem).wait()
        pltpu.sync_copy(rows_v, out_hbm.at[pl.ds(base, per_w)])

    return k(table, idx)


def kernel(x, x_xyz, bin_tokens, Wq, Wk, Wv):
    Bb, Cc, Nn = x.shape
    tokens = jnp.broadcast_to(bin_tokens, (Bb, Cc, NUM_BINS))
    x_and_token = jnp.concatenate([x, tokens], axis=-1)
    q = jnp.einsum('oc,bcn->bon', Wq, x_and_token)
    k = jnp.einsum('oc,bcn->bon', Wk, x_and_token)
    d = q.shape[1]
    energy = jnp.einsum('bdn,bdm->bnm', q, k) / jnp.sqrt(jnp.float32(d))
    attention = jax.nn.softmax(energy, axis=-1)
    aps = _fused_aps_pallas(attention, x_xyz)
    token_scores = jnp.sum(attention[:, Nn:, :Nn], axis=-1)
    bin_prob = jax.nn.softmax(token_scores, axis=-1)
    ks = jnp.floor((2 * M / NUM_BINS) * bin_prob).astype(jnp.int32)
    ks = jnp.clip(ks, 0, CS)
    last = jnp.clip(M - jnp.sum(ks[:, :-1], axis=-1), 0, CS)
    ks = jnp.concatenate([ks[:, :-1], last[:, None]], axis=-1)
    kscum = jnp.cumsum(ks, axis=-1)
    kscum16 = jnp.concatenate(
        [jnp.zeros((Bb, 1), jnp.int32), kscum,
         jnp.zeros((Bb, 7), jnp.int32)], axis=-1)[:, None, :]
    idx_batch = _select_pallas(aps, kscum16)
    vT = jnp.einsum('oc,bcn->bno', Wv, x_and_token[:, :, :Nn])
    table = vT.reshape(Bb * Nn, Q_OUT)
    glob_idx = (idx_batch
                + (jnp.arange(Bb, dtype=jnp.int32) * Nn)[:, None]).reshape(-1)
    rows = _sc_gather_rows(table, glob_idx)
    x_ds = jnp.swapaxes(rows.reshape(Bb, M, Q_OUT), 1, 2)
    return x_ds, idx_batch


# speedup vs baseline: 1.4880x; 1.4880x over previous
import functools

import jax
import jax.numpy as jnp
from jax import lax
from jax.experimental import pallas as pl
from jax.experimental.pallas import tpu as pltpu
from jax.experimental.pallas import tpu_sc as plsc

B, C, N = 4, 256, 2048
NUM_BINS = 8
M = 256
K_NN = 32
Q_OUT = 256
CS = N // NUM_BINS
NT = N + NUM_BINS


def _sortable_key(f):
    u = lax.bitcast_convert_type(f, jnp.int32)
    int_min = jnp.asarray(-2147483648, jnp.int32)
    return jnp.where(u >= 0, u, (int_min - u) - 1)


def _fused_body(attn_ref, xyzt_ref, xyz_ref, sqc_ref, sqr_ref, acc_ref):
    i = pl.program_id(1)
    inner = lax.dot_general(xyzt_ref[0], xyz_ref[0], (((1,), (0,)), ((), ())),
                            preferred_element_type=jnp.float32)
    d2b = (sqc_ref[0] + sqr_ref[0]) - 2.0 * inner
    key = _sortable_key(d2b)
    R = key.shape[0]
    cols = lax.broadcasted_iota(jnp.int32, (R, N), 1)
    g = lax.broadcasted_iota(jnp.int32, (R, 1), 0) + i * R
    int_max = jnp.asarray(2147483647, jnp.int32)
    keyx = jnp.where(cols == g, int_max, key)
    lo = jnp.min(keyx, axis=1, keepdims=True)
    m128 = key[:, 0:128]
    for t in range(1, 16):
        m128 = jnp.minimum(m128, key[:, t * 128:(t + 1) * 128])
    hi = jnp.max(m128, axis=1, keepdims=True)
    hi = jnp.maximum(hi, lo)

    def bs_cond(carry):
        lo, hi = carry
        return jnp.any(lo < hi)

    def bs_body(carry):
        lo, hi = carry
        mid = (lo & hi) + ((lo ^ hi) >> 1)
        cnt = jnp.sum((key <= mid).astype(jnp.int32), axis=1, keepdims=True)
        ge = cnt >= K_NN
        return jnp.where(ge, lo, mid + 1), jnp.where(ge, mid, hi)

    lo, hi = lax.while_loop(bs_cond, bs_body, (lo, hi))
    t = hi
    cnt_t = jnp.sum((key <= t).astype(jnp.int32), axis=1, keepdims=True)

    def with_ties(_):
        c_less = jnp.sum((key < t).astype(jnp.int32), axis=1, keepdims=True)
        t_allowed = K_NN - c_less
        tie = key == t
        tlo = jnp.zeros((R, 1), jnp.int32)
        thi = jnp.full((R, 1), jnp.int32(N - 1))

        def ts_body(_, carry):
            tlo, thi = carry
            mid = tlo + ((thi - tlo) >> 1)
            cnt = jnp.sum((tie & (cols <= mid)).astype(jnp.int32), axis=1,
                          keepdims=True)
            ge = cnt >= t_allowed
            return jnp.where(ge, tlo, mid + 1), jnp.where(ge, mid, thi)

        _, thi = lax.fori_loop(0, 11, ts_body, (tlo, thi))
        return thi

    def no_ties(_):
        return jnp.full((R, 1), jnp.int32(N - 1))

    T = lax.cond(jnp.any(cnt_t > K_NN), with_ties, no_ties, 0)
    maskf = ((key < t) | ((key == t) & (cols <= T))).astype(jnp.float32)
    prod = attn_ref[0, :, :N] * maskf

    @pl.when(i == 0)
    def _():
        acc_ref[...] = jnp.zeros_like(acc_ref)

    acc = acc_ref[0, 0]
    for t2 in range(R // 8):
        acc = acc + prod[t2 * 8:(t2 + 1) * 8, :]
    acc_ref[0, 0] = acc


def _fused_aps_pallas(attn, x_xyz):
    sq = jnp.sum(x_xyz * x_xyz, axis=1)
    xyzT = jnp.swapaxes(x_xyz, 1, 2)
    RB = 512
    acc8 = pl.pallas_call(
        _fused_body,
        grid=(B, N // RB),
        in_specs=[pl.BlockSpec((1, RB, NT), lambda b, i: (b, i, 0)),
                  pl.BlockSpec((1, RB, 3), lambda b, i: (b, i, 0)),
                  pl.BlockSpec((1, 3, N), lambda b, i: (b, 0, 0)),
                  pl.BlockSpec((1, RB, 1), lambda b, i: (b, i, 0)),
                  pl.BlockSpec((1, 1, N), lambda b, i: (b, 0, 0))],
        out_specs=pl.BlockSpec((1, 1, 8, N), lambda b, i: (b, 0, 0, 0)),
        out_shape=jax.ShapeDtypeStruct((B, 1, 8, N), jnp.float32),
    )(attn, xyzT, x_xyz, sq[:, :, None], sq[:, None, :])[:, 0]
    s = acc8
    t_ = s[:, 0:4] + s[:, 4:8]
    t_ = t_[:, 0:2] + t_[:, 2:4]
    return t_[:, 0] + t_[:, 1]


def _select_body(aps_row_ref, aps_col_ref, kscum_ref, idx_ref):
    j = pl.program_id(1)
    a_row = aps_row_ref[0]
    a_col = aps_col_ref[0]
    ksrow = kscum_ref[0]
    RB = a_col.shape[0]
    cols = lax.broadcasted_iota(jnp.int32, (RB, N), 1)
    rows = lax.broadcasted_iota(jnp.int32, (RB, 1), 0)
    m_col = rows + j * RB
    gt = (a_row > a_col).astype(jnp.int32)
    tie = ((a_row == a_col) & (cols < m_col)).astype(jnp.int32)
    p = jnp.sum(gt + tie, axis=1, keepdims=True)
    j_bin = p >> 8
    pos = p & jnp.int32(CS - 1)
    t16 = lax.broadcasted_iota(jnp.int32, (RB, 16), 1)
    kscum_j = jnp.sum(jnp.where(t16 == j_bin, ksrow, 0), axis=1, keepdims=True)
    kscum_j1 = jnp.sum(jnp.where(t16 == j_bin + 1, ksrow, 0), axis=1,
                       keepdims=True)
    flag = pos < (kscum_j1 - kscum_j)
    slot = kscum_j + pos
    total = jnp.sum(jnp.where(lax.broadcasted_iota(jnp.int32, (1, 16), 1) == 8,
                              kscum_ref[0], 0), axis=1, keepdims=True)
    s_iota = lax.broadcasted_iota(jnp.int32, (RB, M), 1)
    sel = flag & (s_iota == slot)
    pad = s_iota == (p + total)
    contrib = jnp.sum(m_col * (sel.astype(jnp.int32) + pad.astype(jnp.int32)),
                      axis=0, keepdims=True)

    @pl.when(j == 0)
    def _():
        idx_ref[0] = jnp.zeros_like(idx_ref[0])

    idx_ref[0] += contrib


def _select_pallas(aps, kscum16):
    RB = 256
    aps_row = aps[:, None, :]
    aps_col = aps[:, :, None]
    idx = pl.pallas_call(
        _select_body,
        grid=(B, N // RB),
        in_specs=[
            pl.BlockSpec((1, 1, N), lambda b, i: (b, 0, 0)),
            pl.BlockSpec((1, RB, 1), lambda b, i: (b, i, 0)),
            pl.BlockSpec((1, 1, 16), lambda b, i: (b, 0, 0)),
        ],
        out_specs=pl.BlockSpec((1, 1, M), lambda b, i: (b, 0, 0)),
        out_shape=jax.ShapeDtypeStruct((B, 1, M), jnp.int32),
    )(aps_row, aps_col, kscum16)
    return idx[:, 0, :]


def _sc_gather_rows(table, idx):
    n_rows, depth = idx.shape[0], table.shape[1]
    info = plsc.get_sparse_core_info()
    nw = info.num_cores * info.num_subcores
    per_w = n_rows // nw
    mesh = plsc.VectorSubcoreMesh(core_axis_name="c", subcore_axis_name="s")

    @functools.partial(
        pl.kernel, mesh=mesh,
        out_type=jax.ShapeDtypeStruct((n_rows, depth), jnp.float32),
        scratch_types=[
            pltpu.VMEM((per_w,), jnp.int32),
            pltpu.VMEM((per_w, depth), jnp.float32),
            pltpu.SemaphoreType.DMA,
        ],
    )
    def k(table_hbm, idx_hbm, out_hbm, idx_v, rows_v, sem):
        wid = lax.axis_index("s") * info.num_cores + lax.axis_index("c")
        base = wid * per_w
        pltpu.sync_copy(idx_hbm.at[pl.ds(base, per_w)], idx_v)
        pltpu.async_copy(table_hbm.at[idx_v], rows_v, sem).wait()
        pltpu.sync_copy(rows_v, out_hbm.at[pl.ds(base, per_w)])

    return k(table, idx)


def kernel(x, x_xyz, bin_tokens, Wq, Wk, Wv):
    Bb, Cc, Nn = x.shape
    tokens = jnp.broadcast_to(bin_tokens, (Bb, Cc, NUM_BINS))
    x_and_token = jnp.concatenate([x, tokens], axis=-1)
    q = jnp.einsum('oc,bcn->bon', Wq, x_and_token)
    k = jnp.einsum('oc,bcn->bon', Wk, x_and_token)
    d = q.shape[1]
    energy = jnp.einsum('bdn,bdm->bnm', q, k) / jnp.sqrt(jnp.float32(d))
    attention = jax.nn.softmax(energy, axis=-1)
    aps = _fused_aps_pallas(attention, x_xyz)
    token_scores = jnp.sum(attention[:, Nn:, :Nn], axis=-1)
    bin_prob = jax.nn.softmax(token_scores, axis=-1)
    ks = jnp.floor((2 * M / NUM_BINS) * bin_prob).astype(jnp.int32)
    ks = jnp.clip(ks, 0, CS)
    last = jnp.clip(M - jnp.sum(ks[:, :-1], axis=-1), 0, CS)
    ks = jnp.concatenate([ks[:, :-1], last[:, None]], axis=-1)
    kscum = jnp.cumsum(ks, axis=-1)
    kscum16 = jnp.concatenate(
        [jnp.zeros((Bb, 1), jnp.int32), kscum,
         jnp.zeros((Bb, 7), jnp.int32)], axis=-1)[:, None, :]
    idx_batch = _select_pallas(aps, kscum16)
    vT = jnp.einsum('oc,bcn->bno', Wv, x_and_token[:, :, :Nn])
    table = vT.reshape(Bb * Nn, Q_OUT)
    glob_idx = (idx_batch
                + (jnp.arange(Bb, dtype=jnp.int32) * Nn)[:, None]).reshape(-1)
    rows = _sc_gather_rows(table, glob_idx)
    x_ds = jnp.swapaxes(rows.reshape(Bb, M, Q_OUT), 1, 2)
    return x_ds, idx_batch


# speedup vs baseline: 1.5375x; 1.0332x over previous
import functools

import jax
import jax.numpy as jnp
from jax import lax
from jax.experimental import pallas as pl
from jax.experimental.pallas import tpu as pltpu
from jax.experimental.pallas import tpu_sc as plsc

B, C, N = 4, 256, 2048
NUM_BINS = 8
M = 256
K_NN = 32
Q_OUT = 256
CS = N // NUM_BINS
NT = N + NUM_BINS


def _sortable_key(f):
    u = lax.bitcast_convert_type(f, jnp.int32)
    int_min = jnp.asarray(-2147483648, jnp.int32)
    return jnp.where(u >= 0, u, (int_min - u) - 1)


def _fused_body(attn_ref, xyzt_ref, xyz_ref, sqc_ref, sqr_ref, acc_ref):
    i = pl.program_id(1)
    inner = lax.dot_general(xyzt_ref[0], xyz_ref[0], (((1,), (0,)), ((), ())),
                            preferred_element_type=jnp.float32)
    d2b = (sqc_ref[0] + sqr_ref[0]) - 2.0 * inner
    key = _sortable_key(d2b)
    R = key.shape[0]
    cols = lax.broadcasted_iota(jnp.int32, (R, N), 1)
    g = lax.broadcasted_iota(jnp.int32, (R, 1), 0) + i * R
    int_max = jnp.asarray(2147483647, jnp.int32)
    keyx = jnp.where(cols == g, int_max, key)
    lo = jnp.min(keyx, axis=1, keepdims=True)
    m128 = key[:, 0:128]
    for t in range(1, 16):
        m128 = jnp.minimum(m128, key[:, t * 128:(t + 1) * 128])
    hi = jnp.max(m128, axis=1, keepdims=True)
    hi = jnp.maximum(hi, lo)

    def bs_cond(carry):
        lo, hi = carry
        return jnp.any(lo < hi)

    def bs_body(carry):
        lo, hi = carry
        mid = (lo & hi) + ((lo ^ hi) >> 1)
        cnt = jnp.sum((key <= mid).astype(jnp.int32), axis=1, keepdims=True)
        ge = cnt >= K_NN
        return jnp.where(ge, lo, mid + 1), jnp.where(ge, mid, hi)

    lo, hi = lax.while_loop(bs_cond, bs_body, (lo, hi))
    t = hi
    cnt_t = jnp.sum((key <= t).astype(jnp.int32), axis=1, keepdims=True)

    def with_ties(_):
        c_less = jnp.sum((key < t).astype(jnp.int32), axis=1, keepdims=True)
        t_allowed = K_NN - c_less
        tie = key == t
        tlo = jnp.zeros((R, 1), jnp.int32)
        thi = jnp.full((R, 1), jnp.int32(N - 1))

        def ts_body(_, carry):
            tlo, thi = carry
            mid = tlo + ((thi - tlo) >> 1)
            cnt = jnp.sum((tie & (cols <= mid)).astype(jnp.int32), axis=1,
                          keepdims=True)
            ge = cnt >= t_allowed
            return jnp.where(ge, tlo, mid + 1), jnp.where(ge, mid, thi)

        _, thi = lax.fori_loop(0, 11, ts_body, (tlo, thi))
        return thi

    def no_ties(_):
        return jnp.full((R, 1), jnp.int32(N - 1))

    T = lax.cond(jnp.any(cnt_t > K_NN), with_ties, no_ties, 0)
    maskf = ((key < t) | ((key == t) & (cols <= T))).astype(jnp.float32)
    prod = attn_ref[0, :, :N] * maskf

    @pl.when(i == 0)
    def _():
        acc_ref[...] = jnp.zeros_like(acc_ref)

    acc = acc_ref[0, 0]
    for t2 in range(R // 8):
        acc = acc + prod[t2 * 8:(t2 + 1) * 8, :]
    acc_ref[0, 0] = acc


def _fused_aps_pallas(attn, x_xyz):
    sq = jnp.sum(x_xyz * x_xyz, axis=1)
    xyzT = jnp.swapaxes(x_xyz, 1, 2)
    RB = 1024
    acc8 = pl.pallas_call(
        _fused_body,
        grid=(B, N // RB),
        in_specs=[pl.BlockSpec((1, RB, NT), lambda b, i: (b, i, 0)),
                  pl.BlockSpec((1, RB, 3), lambda b, i: (b, i, 0)),
                  pl.BlockSpec((1, 3, N), lambda b, i: (b, 0, 0)),
                  pl.BlockSpec((1, RB, 1), lambda b, i: (b, i, 0)),
                  pl.BlockSpec((1, 1, N), lambda b, i: (b, 0, 0))],
        out_specs=pl.BlockSpec((1, 1, 8, N), lambda b, i: (b, 0, 0, 0)),
        out_shape=jax.ShapeDtypeStruct((B, 1, 8, N), jnp.float32),
    )(attn, xyzT, x_xyz, sq[:, :, None], sq[:, None, :])[:, 0]
    s = acc8
    t_ = s[:, 0:4] + s[:, 4:8]
    t_ = t_[:, 0:2] + t_[:, 2:4]
    return t_[:, 0] + t_[:, 1]


def _select_body(aps_row_ref, aps_col_ref, kscum_ref, idx_ref):
    j = pl.program_id(1)
    a_row = aps_row_ref[0]
    a_col = aps_col_ref[0]
    ksrow = kscum_ref[0]
    RB = a_col.shape[0]
    cols = lax.broadcasted_iota(jnp.int32, (RB, N), 1)
    rows = lax.broadcasted_iota(jnp.int32, (RB, 1), 0)
    m_col = rows + j * RB
    gt = (a_row > a_col).astype(jnp.int32)
    tie = ((a_row == a_col) & (cols < m_col)).astype(jnp.int32)
    p = jnp.sum(gt + tie, axis=1, keepdims=True)
    j_bin = p >> 8
    pos = p & jnp.int32(CS - 1)
    t16 = lax.broadcasted_iota(jnp.int32, (RB, 16), 1)
    kscum_j = jnp.sum(jnp.where(t16 == j_bin, ksrow, 0), axis=1, keepdims=True)
    kscum_j1 = jnp.sum(jnp.where(t16 == j_bin + 1, ksrow, 0), axis=1,
                       keepdims=True)
    flag = pos < (kscum_j1 - kscum_j)
    slot = kscum_j + pos
    total = jnp.sum(jnp.where(lax.broadcasted_iota(jnp.int32, (1, 16), 1) == 8,
                              kscum_ref[0], 0), axis=1, keepdims=True)
    s_iota = lax.broadcasted_iota(jnp.int32, (RB, M), 1)
    sel = flag & (s_iota == slot)
    pad = s_iota == (p + total)
    contrib = jnp.sum(m_col * (sel.astype(jnp.int32) + pad.astype(jnp.int32)),
                      axis=0, keepdims=True)

    @pl.when(j == 0)
    def _():
        idx_ref[0] = jnp.zeros_like(idx_ref[0])

    idx_ref[0] += contrib


def _select_pallas(aps, kscum16):
    RB = 256
    aps_row = aps[:, None, :]
    aps_col = aps[:, :, None]
    idx = pl.pallas_call(
        _select_body,
        grid=(B, N // RB),
        in_specs=[
            pl.BlockSpec((1, 1, N), lambda b, i: (b, 0, 0)),
            pl.BlockSpec((1, RB, 1), lambda b, i: (b, i, 0)),
            pl.BlockSpec((1, 1, 16), lambda b, i: (b, 0, 0)),
        ],
        out_specs=pl.BlockSpec((1, 1, M), lambda b, i: (b, 0, 0)),
        out_shape=jax.ShapeDtypeStruct((B, 1, M), jnp.int32),
    )(aps_row, aps_col, kscum16)
    return idx[:, 0, :]


def _sc_gather_rows(table, idx):
    n_rows, depth = idx.shape[0], table.shape[1]
    info = plsc.get_sparse_core_info()
    nw = info.num_cores * info.num_subcores
    per_w = n_rows // nw
    mesh = plsc.VectorSubcoreMesh(core_axis_name="c", subcore_axis_name="s")

    @functools.partial(
        pl.kernel, mesh=mesh,
        out_type=jax.ShapeDtypeStruct((n_rows, depth), jnp.float32),
        scratch_types=[
            pltpu.VMEM((per_w,), jnp.int32),
            pltpu.VMEM((per_w, depth), jnp.float32),
            pltpu.SemaphoreType.DMA,
        ],
    )
    def k(table_hbm, idx_hbm, out_hbm, idx_v, rows_v, sem):
        wid = lax.axis_index("s") * info.num_cores + lax.axis_index("c")
        base = wid * per_w
        pltpu.sync_copy(idx_hbm.at[pl.ds(base, per_w)], idx_v)
        pltpu.async_copy(table_hbm.at[idx_v], rows_v, sem).wait()
        pltpu.sync_copy(rows_v, out_hbm.at[pl.ds(base, per_w)])

    return k(table, idx)


def kernel(x, x_xyz, bin_tokens, Wq, Wk, Wv):
    Bb, Cc, Nn = x.shape
    tokens = jnp.broadcast_to(bin_tokens, (Bb, Cc, NUM_BINS))
    x_and_token = jnp.concatenate([x, tokens], axis=-1)
    q = jnp.einsum('oc,bcn->bon', Wq, x_and_token)
    k = jnp.einsum('oc,bcn->bon', Wk, x_and_token)
    d = q.shape[1]
    energy = jnp.einsum('bdn,bdm->bnm', q, k) / jnp.sqrt(jnp.float32(d))
    attention = jax.nn.softmax(energy, axis=-1)
    aps = _fused_aps_pallas(attention, x_xyz)
    token_scores = jnp.sum(attention[:, Nn:, :Nn], axis=-1)
    bin_prob = jax.nn.softmax(token_scores, axis=-1)
    ks = jnp.floor((2 * M / NUM_BINS) * bin_prob).astype(jnp.int32)
    ks = jnp.clip(ks, 0, CS)
    last = jnp.clip(M - jnp.sum(ks[:, :-1], axis=-1), 0, CS)
    ks = jnp.concatenate([ks[:, :-1], last[:, None]], axis=-1)
    kscum = jnp.cumsum(ks, axis=-1)
    kscum16 = jnp.concatenate(
        [jnp.zeros((Bb, 1), jnp.int32), kscum,
         jnp.zeros((Bb, 7), jnp.int32)], axis=-1)[:, None, :]
    idx_batch = _select_pallas(aps, kscum16)
    vT = jnp.einsum('oc,bcn->bno', Wv, x_and_token[:, :, :Nn])
    table = vT.reshape(Bb * Nn, Q_OUT)
    glob_idx = (idx_batch
                + (jnp.arange(Bb, dtype=jnp.int32) * Nn)[:, None]).reshape(-1)
    rows = _sc_gather_rows(table, glob_idx)
    x_ds = jnp.swapaxes(rows.reshape(Bb, M, Q_OUT), 1, 2)
    return x_ds, idx_batch
